# baseline (speedup 1.0000x reference)
import functools

import jax
import jax.numpy as jnp
from jax import lax
from jax.experimental import pallas as pl
from jax.experimental.pallas import tpu as pltpu
from jax.experimental.pallas import tpu_sc as plsc

D_MODEL = 256
T2V_IN = 8
T2V_K = 32

_INV_PI = 0.3183098861837907
_PI = 3.14159265358979
_MAGIC = 12582912.0
_S1 = 0.9999966
_S3 = -0.16664824
_S5 = 0.00830629
_S7 = -0.00018363


def _fast_sin(v):
    t = v * _INV_PI + _MAGIC
    tb = jax.lax.bitcast_convert_type(t, jnp.int32)
    n = (jnp.bitwise_and(tb, 0x7FFFFF) - 0x400000).astype(jnp.float32)
    r = v - n * _PI
    r2 = r * r
    p = (((_S7 * r2 + _S5) * r2 + _S3) * r2 + _S1) * r
    signbit = jnp.left_shift(tb, 31)
    pb = jax.lax.bitcast_convert_type(p, jnp.int32)
    return jax.lax.bitcast_convert_type(jnp.bitwise_xor(pb, signbit),
                                        jnp.float32)


def _fused_body(y_ref, x_ref, wm_ref, bflat_ref, table_ref, wtv_ref, btv_ref,
                linmask_ref, linmaskc_ref, iota_ref, out_ref, *, blk, seq_len):
    yb = y_ref[...]

    pid = pl.program_id(0)
    blocks_per_seq = seq_len // blk
    l_start = (pid % blocks_per_seq) * blk
    lp = (jax.lax.broadcasted_iota(jnp.int32, (blk, 1), 0).astype(jnp.float32)
          + jnp.float32(l_start)) * jnp.float32(1.0 / seq_len)

    xx = jnp.concatenate([x_ref[...], lp], axis=1)
    val = (jnp.dot(xx, wm_ref[...], preferred_element_type=jnp.float32)
           + bflat_ref[...])
    t2v = _fast_sin(val) * linmaskc_ref[...] + val * linmask_ref[...]

    one = jnp.bfloat16(1.0)
    zero = jnp.bfloat16(0.0)
    src_f = jnp.floor(yb[:, 4:5]).astype(jnp.bfloat16)
    idv_f = jnp.floor(yb[:, 5:6]).astype(jnp.bfloat16) + jnp.bfloat16(128.0)
    evt_f = jnp.floor(yb[:, 6:7]).astype(jnp.bfloat16)
    iota = iota_ref[...]
    oh = (jnp.where(iota == src_f, one, zero)
          + jnp.where(iota == evt_f, one, zero)
          + jnp.where(iota == idv_f, one, zero))
    gathered = jnp.dot(oh, table_ref[...], preferred_element_type=jnp.float32)

    a0 = (wtv_ref[0:1, :] + wtv_ref[2:3, :]
          + wtv_ref[4:5, :] + wtv_ref[6:7, :])
    bsum = (btv_ref[0:1, :] + btv_ref[1:2, :]
            + btv_ref[2:3, :] + btv_ref[3:4, :])
    wtv5 = jnp.concatenate(
        [wtv_ref[1:2, :], wtv_ref[3:4, :], wtv_ref[5:6, :], wtv_ref[7:8, :],
         a0], axis=0)
    tv = (jnp.dot(yb[:, 0:5], wtv5, preferred_element_type=jnp.float32)
          + bsum)

    out_ref[...] = t2v + gathered + tv


_SC_WORKERS = 32
_SC_TILE = 256


def _sc_zeros(n, d):
    rows = n // _SC_WORKERS

    @functools.partial(
        pl.kernel,
        mesh=plsc.VectorSubcoreMesh(core_axis_name="c", subcore_axis_name="s"),
        out_type=jax.ShapeDtypeStruct((n, d), jnp.float32),
        scratch_types=[pltpu.VMEM((_SC_TILE, d), jnp.float32)],
    )
    def zk(zsrc_hbm, out_hbm, zbuf):
        wid = lax.axis_index("s") * 2 + lax.axis_index("c")
        base = wid * rows
        pltpu.sync_copy(zsrc_hbm, zbuf)
        for j in range(rows // _SC_TILE):
            pltpu.sync_copy(zbuf, out_hbm.at[pl.ds(base + j * _SC_TILE,
                                                   _SC_TILE)])

    return zk(jnp.zeros((_SC_TILE, d), jnp.float32))


@jax.jit
def kernel(y, x, t2v_w, t2v_b, te_table, id_table,
           w0, b0, w1, b1, w2, b2, w3, b3):
    bs, L, _ = y.shape
    n = bs * L
    blk = 4096
    grid = n // blk

    yf = y.reshape(n, 7)
    xf = x.reshape(n, 7)

    wm = (jnp.eye(T2V_IN, dtype=jnp.float32)[:, :, None]
          * t2v_w[None, :, :]).reshape(T2V_IN, D_MODEL)
    bflat = t2v_b.reshape(1, D_MODEL)
    table = jnp.zeros((2 * 128, D_MODEL), jnp.float32)
    table = table.at[:te_table.shape[0], :].set(te_table)
    table = table.at[128:128 + id_table.shape[0], :].set(id_table)
    table = table.astype(jnp.bfloat16)
    wtv = jnp.concatenate([w0, w1, w2, w3], axis=0)
    btv = jnp.stack([b0, b1, b2, b3], axis=0)
    linmask = (jnp.arange(D_MODEL, dtype=jnp.int32) % T2V_K == 0
               ).astype(jnp.float32).reshape(1, D_MODEL)
    linmaskc = 1.0 - linmask
    iota = jnp.arange(2 * 128, dtype=jnp.float32
                      ).astype(jnp.bfloat16).reshape(1, 2 * 128)

    body = functools.partial(_fused_body, blk=blk, seq_len=L)
    emb = pl.pallas_call(
        body,
        grid=(grid,),
        in_specs=[
            pl.BlockSpec((blk, 7), lambda g: (g, 0)),
            pl.BlockSpec((blk, 7), lambda g: (g, 0)),
            pl.BlockSpec((T2V_IN, D_MODEL), lambda g: (0, 0)),
            pl.BlockSpec((1, D_MODEL), lambda g: (0, 0)),
            pl.BlockSpec((2 * 128, D_MODEL), lambda g: (0, 0)),
            pl.BlockSpec((8, D_MODEL), lambda g: (0, 0)),
            pl.BlockSpec((4, D_MODEL), lambda g: (0, 0)),
            pl.BlockSpec((1, D_MODEL), lambda g: (0, 0)),
            pl.BlockSpec((1, D_MODEL), lambda g: (0, 0)),
            pl.BlockSpec((1, 2 * 128), lambda g: (0, 0)),
        ],
        out_specs=pl.BlockSpec((blk, D_MODEL), lambda g: (g, 0)),
        out_shape=jax.ShapeDtypeStruct((n, D_MODEL), jnp.float32),
    )(yf, xf, wm, bflat, table, wtv, btv, linmask, linmaskc, iota)

    zeros = _sc_zeros(n, D_MODEL)
    emb = emb.reshape(bs, L, D_MODEL)
    return (emb, zeros.reshape(bs, L, D_MODEL))


# speedup vs baseline: 1.1491x; 1.1491x over previous
import functools

import jax
import jax.numpy as jnp
from jax.experimental import pallas as pl

D_MODEL = 256
T2V_IN = 8
T2V_K = 32

_INV_PI = 0.3183098861837907
_PI = 3.14159265358979
_MAGIC = 12582912.0
_S1 = 0.9999966
_S3 = -0.16664824
_S5 = 0.00830629
_S7 = -0.00018363


def _fast_sin(v):
    t = v * _INV_PI + _MAGIC
    tb = jax.lax.bitcast_convert_type(t, jnp.int32)
    n = (jnp.bitwise_and(tb, 0x7FFFFF) - 0x400000).astype(jnp.float32)
    r = v - n * _PI
    r2 = r * r
    p = (((_S7 * r2 + _S5) * r2 + _S3) * r2 + _S1) * r
    signbit = jnp.left_shift(tb, 31)
    pb = jax.lax.bitcast_convert_type(p, jnp.int32)
    return jax.lax.bitcast_convert_type(jnp.bitwise_xor(pb, signbit),
                                        jnp.float32)


def _fused_body(y_ref, x_ref, wm_ref, bflat_ref, table_ref, wtv_ref, btv_ref,
                linmask_ref, linmaskc_ref, iota_ref, out_ref, zeros_ref, *, blk, seq_len):
    yb = y_ref[...]

    pid = pl.program_id(0)
    blocks_per_seq = seq_len // blk
    l_start = (pid % blocks_per_seq) * blk
    lp = (jax.lax.broadcasted_iota(jnp.int32, (blk, 1), 0).astype(jnp.float32)
          + jnp.float32(l_start)) * jnp.float32(1.0 / seq_len)

    xx = jnp.concatenate([x_ref[...], lp], axis=1)
    val = (jnp.dot(xx, wm_ref[...], preferred_element_type=jnp.float32)
           + bflat_ref[...])
    t2v = _fast_sin(val) * linmaskc_ref[...] + val * linmask_ref[...]

    one = jnp.bfloat16(1.0)
    zero = jnp.bfloat16(0.0)
    src_f = jnp.floor(yb[:, 4:5]).astype(jnp.bfloat16)
    idv_f = jnp.floor(yb[:, 5:6]).astype(jnp.bfloat16) + jnp.bfloat16(128.0)
    evt_f = jnp.floor(yb[:, 6:7]).astype(jnp.bfloat16)
    iota = iota_ref[...]
    oh = (jnp.where(iota == src_f, one, zero)
          + jnp.where(iota == evt_f, one, zero)
          + jnp.where(iota == idv_f, one, zero))
    gathered = jnp.dot(oh, table_ref[...], preferred_element_type=jnp.float32)

    a0 = (wtv_ref[0:1, :] + wtv_ref[2:3, :]
          + wtv_ref[4:5, :] + wtv_ref[6:7, :])
    bsum = (btv_ref[0:1, :] + btv_ref[1:2, :]
            + btv_ref[2:3, :] + btv_ref[3:4, :])
    wtv5 = jnp.concatenate(
        [wtv_ref[1:2, :], wtv_ref[3:4, :], wtv_ref[5:6, :], wtv_ref[7:8, :],
         a0], axis=0)
    tv = (jnp.dot(yb[:, 0:5], wtv5, preferred_element_type=jnp.float32)
          + bsum)

    out_ref[...] = t2v + gathered + tv
    zeros_ref[...] = jnp.zeros((blk, D_MODEL), jnp.float32)


@jax.jit
def kernel(y, x, t2v_w, t2v_b, te_table, id_table,
           w0, b0, w1, b1, w2, b2, w3, b3):
    bs, L, _ = y.shape
    n = bs * L
    blk = 4096
    grid = n // blk

    yf = y.reshape(n, 7)
    xf = x.reshape(n, 7)

    wm = (jnp.eye(T2V_IN, dtype=jnp.float32)[:, :, None]
          * t2v_w[None, :, :]).reshape(T2V_IN, D_MODEL)
    bflat = t2v_b.reshape(1, D_MODEL)
    table = jnp.zeros((2 * 128, D_MODEL), jnp.float32)
    table = table.at[:te_table.shape[0], :].set(te_table)
    table = table.at[128:128 + id_table.shape[0], :].set(id_table)
    table = table.astype(jnp.bfloat16)
    wtv = jnp.concatenate([w0, w1, w2, w3], axis=0)
    btv = jnp.stack([b0, b1, b2, b3], axis=0)
    linmask = (jnp.arange(D_MODEL, dtype=jnp.int32) % T2V_K == 0
               ).astype(jnp.float32).reshape(1, D_MODEL)
    linmaskc = 1.0 - linmask
    iota = jnp.arange(2 * 128, dtype=jnp.float32
                      ).astype(jnp.bfloat16).reshape(1, 2 * 128)

    body = functools.partial(_fused_body, blk=blk, seq_len=L)
    emb = pl.pallas_call(
        body,
        grid=(grid,),
        in_specs=[
            pl.BlockSpec((blk, 7), lambda g: (g, 0)),
            pl.BlockSpec((blk, 7), lambda g: (g, 0)),
            pl.BlockSpec((T2V_IN, D_MODEL), lambda g: (0, 0)),
            pl.BlockSpec((1, D_MODEL), lambda g: (0, 0)),
            pl.BlockSpec((2 * 128, D_MODEL), lambda g: (0, 0)),
            pl.BlockSpec((8, D_MODEL), lambda g: (0, 0)),
            pl.BlockSpec((4, D_MODEL), lambda g: (0, 0)),
            pl.BlockSpec((1, D_MODEL), lambda g: (0, 0)),
            pl.BlockSpec((1, D_MODEL), lambda g: (0, 0)),
            pl.BlockSpec((1, 2 * 128), lambda g: (0, 0)),
        ],
        out_specs=[pl.BlockSpec((blk, D_MODEL), lambda g: (g, 0)),
                   pl.BlockSpec((blk, D_MODEL), lambda g: (g, 0))],
        out_shape=[jax.ShapeDtypeStruct((n, D_MODEL), jnp.float32),
                   jax.ShapeDtypeStruct((n, D_MODEL), jnp.float32)],
    )(yf, xf, wm, bflat, table, wtv, btv, linmask, linmaskc, iota)

    emb, zeros = emb
    emb = emb.reshape(bs, L, D_MODEL)
    return (emb, zeros.reshape(bs, L, D_MODEL))
